# baseline (speedup 1.0000x reference)
import functools

import jax
import jax.numpy as jnp
from jax import lax
from jax.experimental import pallas as pl
from jax.experimental.pallas import tpu as pltpu
from jax.experimental.pallas import tpu_sc as plsc

TEMP = 0.05


def _sc_gather_rows(features, targets):
    n_rows, d = features.shape
    b = targets.shape[0]
    try:
        info = plsc.get_sparse_core_info()
        nc, ns = info.num_cores, info.num_subcores
    except Exception:
        nc, ns = 2, 16
    nw = nc * ns
    b_per_w = b // nw
    mesh = plsc.VectorSubcoreMesh(core_axis_name="c", subcore_axis_name="s")

    @functools.partial(
        pl.kernel,
        mesh=mesh,
        out_type=jax.ShapeDtypeStruct((b, d), jnp.float32),
        scratch_types=[
            pltpu.VMEM((b_per_w,), jnp.int32),
            pltpu.VMEM((b_per_w, d), jnp.float32),
            pltpu.SemaphoreType.DMA,
        ],
    )
    def gather_kernel(table_hbm, idx_hbm, out_hbm, idx_v, rows_v, sem):
        wid = lax.axis_index("s") * nc + lax.axis_index("c")
        base = wid * b_per_w
        pltpu.sync_copy(idx_hbm.at[pl.ds(base, b_per_w)], idx_v)
        pltpu.async_copy(table_hbm.at[idx_v], rows_v, sem).wait()
        pltpu.sync_copy(rows_v, out_hbm.at[pl.ds(base, b_per_w)])

    return gather_kernel(features, targets)


def _tc_loss(inputs, features, tgt_rows):
    b, d = inputs.shape
    n = features.shape[0]
    nstreams = 5
    chunk = 2000
    grid = n // (nstreams * chunk)

    log2e = 1.4426950408889634

    def body(x_ref, *rest):
        f_refs = rest[:nstreams]
        t_ref = rest[nstreams]
        out_ref = rest[nstreams + 1]
        xs_ref, xsb_ref, acc_ref = rest[nstreams + 2:]
        i = pl.program_id(0)

        @pl.when(i == 0)
        def _prologue():
            x = x_ref[...]
            nrm = jnp.sum(x * x, axis=1, keepdims=True)
            xs = x * (1.0 / (jnp.sqrt(nrm) * TEMP))
            xs_ref[...] = xs
            xsb_ref[...] = (xs * log2e).astype(jnp.bfloat16)
            acc_ref[...] = jnp.zeros_like(acc_ref)

        part = acc_ref[...]
        for f_ref in f_refs:
            s2 = lax.dot_general(
                xsb_ref[...], f_ref[...].astype(jnp.bfloat16),
                (((1,), (1,)), ((), ())),
                preferred_element_type=jnp.float32,
            )
            part = part + jnp.sum(jnp.exp2(s2), axis=1, keepdims=True)
        acc_ref[...] = part

        @pl.when(i == grid - 1)
        def _epilogue():
            tgt = jnp.sum(xs_ref[...] * t_ref[...], axis=1, keepdims=True)
            nll = jnp.log(acc_ref[...]) - tgt
            out_ref[0, 0] = jnp.mean(nll)

    f_specs = [
        pl.BlockSpec((chunk, d), functools.partial(lambda k, i: (nstreams * i + k, 0), k))
        for k in range(nstreams)
    ]
    out = pl.pallas_call(
        body,
        grid=(grid,),
        in_specs=[pl.BlockSpec((b, d), lambda i: (0, 0))] + f_specs
        + [pl.BlockSpec((b, d), lambda i: (0, 0))],
        out_specs=pl.BlockSpec((1, 1), lambda i: (0, 0), memory_space=pltpu.SMEM),
        out_shape=jax.ShapeDtypeStruct((1, 1), jnp.float32),
        scratch_shapes=[
            pltpu.VMEM((b, d), jnp.float32),
            pltpu.VMEM((b, d), jnp.bfloat16),
            pltpu.VMEM((b, 1), jnp.float32),
        ],
    )(inputs, *([features] * nstreams), tgt_rows)
    return out[0, 0]


def kernel(inputs, targets, features):
    tgt_rows = _sc_gather_rows(features, targets)
    return _tc_loss(inputs, features, tgt_rows)


# speedup vs baseline: 1.0388x; 1.0388x over previous
import functools

import jax
import jax.numpy as jnp
from jax import lax
from jax.experimental import pallas as pl
from jax.experimental.pallas import tpu as pltpu
from jax.experimental.pallas import tpu_sc as plsc

TEMP = 0.05
LOG2E = 1.4426950408889634


def _sc_gather_rows(features, targets):
    n_rows, d = features.shape
    b = targets.shape[0]
    try:
        info = plsc.get_sparse_core_info()
        nc, ns = info.num_cores, info.num_subcores
    except Exception:
        nc, ns = 2, 16
    nw = nc * ns
    b_per_w = b // nw
    mesh = plsc.VectorSubcoreMesh(core_axis_name="c", subcore_axis_name="s")

    @functools.partial(
        pl.kernel,
        mesh=mesh,
        out_type=jax.ShapeDtypeStruct((b, d), jnp.float32),
        scratch_types=[
            pltpu.VMEM((b_per_w,), jnp.int32),
            pltpu.VMEM((b_per_w, d), jnp.float32),
            pltpu.SemaphoreType.DMA,
        ],
    )
    def gather_kernel(table_hbm, idx_hbm, out_hbm, idx_v, rows_v, sem):
        wid = lax.axis_index("s") * nc + lax.axis_index("c")
        base = wid * b_per_w
        pltpu.sync_copy(idx_hbm.at[pl.ds(base, b_per_w)], idx_v)
        pltpu.async_copy(table_hbm.at[idx_v], rows_v, sem).wait()
        pltpu.sync_copy(rows_v, out_hbm.at[pl.ds(base, b_per_w)])

    return gather_kernel(features, targets)


def _tc_sumexp(inputs, features):
    b, d = inputs.shape
    n = features.shape[0]
    nstreams = 5
    chunk = 2000
    grid = n // (nstreams * chunk)

    def body(x_ref, *rest):
        f_refs = rest[:nstreams]
        acc_ref = rest[nstreams]
        xs_ref = rest[nstreams + 1]
        xsb_ref = rest[nstreams + 2]
        i = pl.program_id(0)

        @pl.when(i == 0)
        def _prologue():
            x = x_ref[...]
            nrm = jnp.sum(x * x, axis=1, keepdims=True)
            xs = x * (LOG2E / (jnp.sqrt(nrm) * TEMP))
            xs_ref[...] = xs
            xsb_ref[...] = xs.astype(jnp.bfloat16)
            acc_ref[...] = jnp.zeros_like(acc_ref)

        part = acc_ref[...]
        for f_ref in f_refs:
            s2 = lax.dot_general(
                xsb_ref[...], f_ref[...].astype(jnp.bfloat16),
                (((1,), (1,)), ((), ())),
                preferred_element_type=jnp.float32,
            )
            part = part + jnp.sum(jnp.exp2(s2), axis=1, keepdims=True)
        acc_ref[...] = part

    f_specs = [
        pl.BlockSpec((chunk, d), functools.partial(lambda k, i: (nstreams * i + k, 0), k))
        for k in range(nstreams)
    ]
    acc, xs = pl.pallas_call(
        body,
        grid=(grid,),
        in_specs=[pl.BlockSpec((b, d), lambda i: (0, 0))] + f_specs,
        out_specs=[
            pl.BlockSpec((b, 1), lambda i: (0, 0)),
            pl.BlockSpec((b, d), lambda i: (0, 0)),
        ],
        out_shape=[
            jax.ShapeDtypeStruct((b, 1), jnp.float32),
            jax.ShapeDtypeStruct((b, d), jnp.float32),
        ],
        scratch_shapes=[
            pltpu.VMEM((b, d), jnp.bfloat16),
        ],
    )(inputs, *([features] * nstreams))
    return acc, xs


def _tc_finish(acc, xs, tgt_rows):
    b, d = xs.shape

    def body(acc_ref, xs_ref, t_ref, out_ref):
        tgt = jnp.sum(xs_ref[...] * t_ref[...], axis=1, keepdims=True)
        nll = jnp.log(acc_ref[...]) - tgt * (1.0 / LOG2E)
        out_ref[0, 0] = jnp.mean(nll)

    out = pl.pallas_call(
        body,
        out_specs=pl.BlockSpec(memory_space=pltpu.SMEM),
        out_shape=jax.ShapeDtypeStruct((1, 1), jnp.float32),
    )(acc, xs, tgt_rows)
    return out[0, 0]


def kernel(inputs, targets, features):
    tgt_rows = _sc_gather_rows(features, targets)
    acc, xs = _tc_sumexp(inputs, features)
    return _tc_finish(acc, xs, tgt_rows)


# speedup vs baseline: 1.0506x; 1.0113x over previous
import functools

import jax
import jax.numpy as jnp
from jax import lax
from jax.experimental import pallas as pl
from jax.experimental.pallas import tpu as pltpu
from jax.experimental.pallas import tpu_sc as plsc

TEMP = 0.05
LOG2E = 1.4426950408889634


def _sc_gather_rows(features, targets):
    n_rows, d = features.shape
    b = targets.shape[0]
    try:
        info = plsc.get_sparse_core_info()
        nc, ns = info.num_cores, info.num_subcores
    except Exception:
        nc, ns = 2, 16
    nw = nc * ns
    b_per_w = b // nw
    mesh = plsc.VectorSubcoreMesh(core_axis_name="c", subcore_axis_name="s")

    @functools.partial(
        pl.kernel,
        mesh=mesh,
        out_type=jax.ShapeDtypeStruct((b, d), jnp.float32),
        scratch_types=[
            pltpu.VMEM((b_per_w,), jnp.int32),
            pltpu.VMEM((b_per_w, d), jnp.float32),
            pltpu.SemaphoreType.DMA,
        ],
    )
    def gather_kernel(table_hbm, idx_hbm, out_hbm, idx_v, rows_v, sem):
        wid = lax.axis_index("s") * nc + lax.axis_index("c")
        base = wid * b_per_w
        pltpu.sync_copy(idx_hbm.at[pl.ds(base, b_per_w)], idx_v)
        pltpu.async_copy(table_hbm.at[idx_v], rows_v, sem).wait()
        pltpu.sync_copy(rows_v, out_hbm.at[pl.ds(base, b_per_w)])

    return gather_kernel(features, targets)


def _tc_sumexp(inputs, features):
    b, d = inputs.shape
    n = features.shape[0]
    nstreams = 5
    chunk = 4000
    grid = n // (nstreams * chunk)

    def body(x_ref, *rest):
        f_refs = rest[:nstreams]
        acc_ref = rest[nstreams]
        xs_ref = rest[nstreams + 1]
        xsb_ref = rest[nstreams + 2]
        i = pl.program_id(0)

        @pl.when(i == 0)
        def _prologue():
            x = x_ref[...]
            nrm = jnp.sum(x * x, axis=1, keepdims=True)
            xs = x * (LOG2E / (jnp.sqrt(nrm) * TEMP))
            xs_ref[...] = xs
            xsb_ref[...] = xs.astype(jnp.bfloat16)
            acc_ref[...] = jnp.zeros_like(acc_ref)

        part = acc_ref[...]
        for f_ref in f_refs:
            s2 = lax.dot_general(
                xsb_ref[...], f_ref[...].astype(jnp.bfloat16),
                (((1,), (1,)), ((), ())),
                preferred_element_type=jnp.float32,
            )
            part = part + jnp.sum(jnp.exp2(s2), axis=1, keepdims=True)
        acc_ref[...] = part

    f_specs = [
        pl.BlockSpec((chunk, d), functools.partial(lambda k, i: (nstreams * i + k, 0), k))
        for k in range(nstreams)
    ]
    acc, xs = pl.pallas_call(
        body,
        grid=(grid,),
        in_specs=[pl.BlockSpec((b, d), lambda i: (0, 0))] + f_specs,
        out_specs=[
            pl.BlockSpec((b, 1), lambda i: (0, 0)),
            pl.BlockSpec((b, d), lambda i: (0, 0)),
        ],
        out_shape=[
            jax.ShapeDtypeStruct((b, 1), jnp.float32),
            jax.ShapeDtypeStruct((b, d), jnp.float32),
        ],
        scratch_shapes=[
            pltpu.VMEM((b, d), jnp.bfloat16),
        ],
    )(inputs, *([features] * nstreams))
    return acc, xs


def _tc_finish(acc, xs, tgt_rows):
    b, d = xs.shape

    def body(acc_ref, xs_ref, t_ref, out_ref):
        tgt = jnp.sum(xs_ref[...] * t_ref[...], axis=1, keepdims=True)
        nll = jnp.log(acc_ref[...]) - tgt * (1.0 / LOG2E)
        out_ref[0, 0] = jnp.mean(nll)

    out = pl.pallas_call(
        body,
        out_specs=pl.BlockSpec(memory_space=pltpu.SMEM),
        out_shape=jax.ShapeDtypeStruct((1, 1), jnp.float32),
    )(acc, xs, tgt_rows)
    return out[0, 0]


def kernel(inputs, targets, features):
    tgt_rows = _sc_gather_rows(features, targets)
    acc, xs = _tc_sumexp(inputs, features)
    return _tc_finish(acc, xs, tgt_rows)
